# baseline (speedup 1.0000x reference)
import functools

import jax
import jax.numpy as jnp
from jax import lax
from jax.experimental import pallas as pl
from jax.experimental.pallas import tpu as pltpu
from jax.experimental.pallas import tpu_sc as plsc

_LANES = 16
_N_CHUNKS = 4


@functools.cache
def _build(B, V, D):
    info = plsc.get_sparse_core_info()
    nc, ns = info.num_cores, info.num_subcores
    nw = nc * ns
    assert B % (nw * _LANES * _N_CHUNKS) == 0
    b_per_w = B // nw
    chunk = b_per_w // _N_CHUNKS

    mesh = plsc.VectorSubcoreMesh(core_axis_name="c", subcore_axis_name="s")

    @functools.partial(
        pl.kernel,
        mesh=mesh,
        out_type=jax.ShapeDtypeStruct((D, B), jnp.float32),
        scratch_types=[
            pltpu.VMEM((b_per_w,), jnp.int32),
            pltpu.VMEM((V, D), jnp.float32),
            pltpu.VMEM((V * D,), jnp.float32),
            pltpu.VMEM((D, b_per_w), jnp.float32),
            pltpu.SemaphoreType.DMA,
            pltpu.SemaphoreType.DMA,
        ],
        compiler_params=pltpu.CompilerParams(
            needs_layout_passes=False,
            disable_semaphore_checks=True,
            skip_device_barrier=True,
        ),
    )
    def emb(idx_hbm, table_hbm, out_hbm, idx_v, table2d_v, table_v, buf, lsem, ssem):
        wid = lax.axis_index("s") * nc + lax.axis_index("c")
        base = wid * b_per_w
        cp_idx = pltpu.async_copy(idx_hbm.at[pl.ds(base, b_per_w)], idx_v, lsem)
        cp_tab = pltpu.async_copy(table_hbm, table2d_v, lsem)
        cp_idx.wait()
        cp_tab.wait()

        iota_v = lax.iota(jnp.int32, _LANES) * V

        @plsc.parallel_loop(0, V, unroll=1)
        def transpose(r):
            for c in range(D // _LANES):
                v = table2d_v[r, pl.ds(c * _LANES, _LANES)]
                plsc.store_scatter(table_v, [iota_v + (c * _LANES * V) + r], v)

        @plsc.parallel_loop(0, b_per_w, step=_LANES, unroll=1)
        def body(i):
            idxv = idx_v[pl.ds(i, _LANES)]
            for d in range(D):
                v = plsc.load_gather(table_v, [idxv + d * V])
                buf[d, pl.ds(i, _LANES)] = v

        pltpu.async_copy(buf, out_hbm.at[:, pl.ds(base, b_per_w)], ssem).wait()

    def run(world_id, table):
        return emb(world_id, table).T

    return run


def kernel(world_id, table):
    B, = world_id.shape
    V, D = table.shape
    return _build(B, V, D)(world_id, table)


# speedup vs baseline: 1.0569x; 1.0569x over previous
import functools

import jax
import jax.numpy as jnp
from jax import lax
from jax.experimental import pallas as pl
from jax.experimental.pallas import tpu as pltpu
from jax.experimental.pallas import tpu_sc as plsc

_LANES = 16
_N_CHUNKS = 4


@functools.cache
def _build(B, V, D):
    info = plsc.get_sparse_core_info()
    nc, ns = info.num_cores, info.num_subcores
    nw = nc * ns
    assert B % (nw * _LANES * _N_CHUNKS) == 0
    b_per_w = B // nw
    chunk = b_per_w // _N_CHUNKS

    mesh = plsc.VectorSubcoreMesh(core_axis_name="c", subcore_axis_name="s")

    @functools.partial(
        pl.kernel,
        mesh=mesh,
        out_type=jax.ShapeDtypeStruct((D, B), jnp.float32),
        scratch_types=[
            pltpu.VMEM((b_per_w,), jnp.int32),
            pltpu.VMEM((V * D,), jnp.float32),
            pltpu.VMEM((D, b_per_w), jnp.float32),
            pltpu.SemaphoreType.DMA,
            pltpu.SemaphoreType.DMA,
        ],
        compiler_params=pltpu.CompilerParams(
            needs_layout_passes=False,
            disable_semaphore_checks=True,
            skip_device_barrier=True,
        ),
    )
    def emb(idx_hbm, table_hbm, out_hbm, idx_v, table_v, buf, lsem, ssem):
        wid = lax.axis_index("s") * nc + lax.axis_index("c")
        base = wid * b_per_w
        cp_idx = pltpu.async_copy(idx_hbm.at[pl.ds(base, b_per_w)], idx_v, lsem)
        cp_tab = pltpu.async_copy(table_hbm, table_v, lsem)
        cp_idx.wait()
        cp_tab.wait()

        @plsc.parallel_loop(0, b_per_w, step=_LANES, unroll=1)
        def body(i):
            idxv = idx_v[pl.ds(i, _LANES)]
            for d in range(D):
                v = plsc.load_gather(table_v, [idxv + d * V])
                buf[d, pl.ds(i, _LANES)] = v

        pltpu.async_copy(buf, out_hbm.at[:, pl.ds(base, b_per_w)], ssem).wait()

    def run(world_id, table):
        return emb(world_id, table.T.reshape(-1)).T

    return run


def kernel(world_id, table):
    B, = world_id.shape
    V, D = table.shape
    return _build(B, V, D)(world_id, table)
